# baseline (speedup 1.0000x reference)
import functools

import jax
import jax.numpy as jnp
from jax import lax
from jax.experimental import pallas as pl
from jax.experimental.pallas import tpu as pltpu
from jax.experimental.pallas import tpu_sc as plsc

_BATCH = 4096
_LEN = 200
_DIM = 128
_NCLS = 2
_VOCAB = 100000
_NWORKERS = 32
_SPW = _BATCH // _NWORKERS
_VBLK = 2048
_VPAD = 102400
_LPAD = 256


def _qmatmul_body(t_ref, w0_ref, w1_ref, q0_ref, q1_ref):
    tb = t_ref[...].astype(jnp.bfloat16)
    q0 = jnp.dot(tb, w0_ref[...].astype(jnp.bfloat16),
                 preferred_element_type=jnp.float32)
    q1 = jnp.dot(tb, w1_ref[...].astype(jnp.bfloat16),
                 preferred_element_type=jnp.float32)
    q0_ref[...] = q0.reshape(_VBLK * _LPAD)
    q1_ref[...] = q1.reshape(_VBLK * _LPAD)


def _qmatmul(table, w0t, w1t):
    return pl.pallas_call(
        _qmatmul_body,
        grid=(_VPAD // _VBLK,),
        in_specs=[
            pl.BlockSpec((_VBLK, _DIM), lambda i: (jnp.minimum(i, _VOCAB // _VBLK), 0)),
            pl.BlockSpec((_DIM, _LPAD), lambda i: (0, 0)),
            pl.BlockSpec((_DIM, _LPAD), lambda i: (0, 0)),
        ],
        out_specs=[
            pl.BlockSpec((_VBLK * _LPAD,), lambda i: (i,)),
            pl.BlockSpec((_VBLK * _LPAD,), lambda i: (i,)),
        ],
        out_shape=[
            jax.ShapeDtypeStruct((_VPAD * _LPAD,), jnp.float32),
            jax.ShapeDtypeStruct((_VPAD * _LPAD,), jnp.float32),
        ],
    )(table, w0t, w1t)


def _sc_body(xT_hbm, q0_hbm, q1_hbm, out_hbm,
             idx_v, qidx_v, rows0_v, rows1_v, *sems8):
    wid = lax.axis_index("s") * 2 + lax.axis_index("c")
    base = wid * _SPW

    pltpu.sync_copy(xT_hbm.at[:, pl.ds(base, _SPW)], idx_v)

    sems = sems8

    def prep_idx(l, u):
        for k in range(8):
            xi = idx_v[l, pl.ds(16 * k, 16)]
            qidx_v[u, pl.ds(16 * k, 16)] = xi * _LPAD + l

    def gcopy0(u):
        return pltpu.make_async_copy(
            q0_hbm.at[qidx_v.at[u]], rows0_v.at[u], sems[u])

    def gcopy1(u):
        return pltpu.make_async_copy(
            q1_hbm.at[qidx_v.at[u]], rows1_v.at[u], sems[u])

    _NBUF = 20
    for u in range(_NBUF):
        prep_idx(u, u)
        gcopy0(u).start()
        gcopy1(u).start()

    def lbody(i, accs):
        for u in range(_NBUF):
            l = _NBUF * i + u
            gcopy0(u).wait()
            gcopy1(u).wait()
            r0 = [rows0_v[u, pl.ds(16 * k, 16)] for k in range(8)]
            r1 = [rows1_v[u, pl.ds(16 * k, 16)] for k in range(8)]

            @pl.when(l + _NBUF < _LEN)
            def _():
                prep_idx(l + _NBUF, u)
                gcopy0(u).start()
                gcopy1(u).start()

            new = []
            for k in range(8):
                new.append(accs[2 * k] + r0[k])
                new.append(accs[2 * k + 1] + r1[k])
            accs = tuple(new)
        return accs

    zero = jnp.zeros((16,), jnp.float32)
    accs = lax.fori_loop(0, _LEN // _NBUF, lbody, (zero,) * 16)

    for k in range(8):
        rows0_v[0, pl.ds(16 * k, 16)] = accs[2 * k]
        rows0_v[1, pl.ds(16 * k, 16)] = accs[2 * k + 1]
    pltpu.sync_copy(rows0_v.at[0], out_hbm.at[0, pl.ds(base, _SPW)])
    pltpu.sync_copy(rows0_v.at[1], out_hbm.at[1, pl.ds(base, _SPW)])


_sc_gather = functools.partial(
    pl.kernel,
    out_type=jax.ShapeDtypeStruct((_NCLS, _BATCH), jnp.float32),
    mesh=plsc.VectorSubcoreMesh(core_axis_name="c", subcore_axis_name="s"),
    scratch_types=[
        pltpu.VMEM((_LEN, _SPW), jnp.int32),
        pltpu.VMEM((20, _SPW), jnp.int32),
        pltpu.VMEM((20, _SPW), jnp.float32),
        pltpu.VMEM((20, _SPW), jnp.float32),
    ] + [pltpu.SemaphoreType.DMA] * 20 + [
    ],
)(_sc_body)


def _logsoftmax_body(z_ref, b_ref, o_ref):
    z = z_ref[...] + b_ref[...]
    m = jnp.max(z, axis=-1, keepdims=True)
    e = jnp.exp(z - m)
    lse = m + jnp.log(jnp.sum(e, axis=-1, keepdims=True))
    o_ref[...] = z - lse


def kernel(x, table, W, b):
    xT = x.T
    Wr = W.reshape(_NCLS, _LEN, _DIM)
    w0t = jnp.pad(Wr[0].T, ((0, 0), (0, _LPAD - _LEN)))
    w1t = jnp.pad(Wr[1].T, ((0, 0), (0, _LPAD - _LEN)))
    q0, q1 = _qmatmul(table, w0t, w1t)
    logits = _sc_gather(xT, q0, q1).T
    return pl.pallas_call(
        _logsoftmax_body,
        out_shape=jax.ShapeDtypeStruct((_BATCH, _NCLS), jnp.float32),
    )(logits, b.reshape(1, _NCLS))


# speedup vs baseline: 1.0449x; 1.0449x over previous
import functools

import jax
import jax.numpy as jnp
from jax import lax
from jax.experimental import pallas as pl
from jax.experimental.pallas import tpu as pltpu
from jax.experimental.pallas import tpu_sc as plsc

_BATCH = 4096
_LEN = 200
_DIM = 128
_NCLS = 2
_VOCAB = 100000
_NWORKERS = 32
_SPW = _BATCH // _NWORKERS
_VBLK = 4096
_VPAD = 102400
_LPAD = 256


def _qmatmul_body(t_ref, w0_ref, w1_ref, q0_ref, q1_ref):
    tb = t_ref[...].astype(jnp.bfloat16)
    q0 = jnp.dot(tb, w0_ref[...].astype(jnp.bfloat16),
                 preferred_element_type=jnp.float32)
    q1 = jnp.dot(tb, w1_ref[...].astype(jnp.bfloat16),
                 preferred_element_type=jnp.float32)
    q0_ref[...] = q0.reshape(_VBLK * _LPAD)
    q1_ref[...] = q1.reshape(_VBLK * _LPAD)


def _qmatmul(table, w0t, w1t):
    return pl.pallas_call(
        _qmatmul_body,
        grid=(_VPAD // _VBLK,),
        in_specs=[
            pl.BlockSpec((_VBLK, _DIM), lambda i: (jnp.minimum(i, _VOCAB // _VBLK), 0)),
            pl.BlockSpec((_DIM, _LPAD), lambda i: (0, 0)),
            pl.BlockSpec((_DIM, _LPAD), lambda i: (0, 0)),
        ],
        out_specs=[
            pl.BlockSpec((_VBLK * _LPAD,), lambda i: (i,)),
            pl.BlockSpec((_VBLK * _LPAD,), lambda i: (i,)),
        ],
        out_shape=[
            jax.ShapeDtypeStruct((_VPAD * _LPAD,), jnp.float32),
            jax.ShapeDtypeStruct((_VPAD * _LPAD,), jnp.float32),
        ],
    )(table, w0t, w1t)


def _sc_body(xT_hbm, q0_hbm, q1_hbm, out_hbm,
             idx_v, qidx_v, rows0_v, rows1_v, *sems8):
    wid = lax.axis_index("s") * 2 + lax.axis_index("c")
    base = wid * _SPW

    pltpu.sync_copy(xT_hbm.at[:, pl.ds(base, _SPW)], idx_v)

    sems = sems8

    def prep_idx(l, u):
        for k in range(8):
            xi = idx_v[l, pl.ds(16 * k, 16)]
            qidx_v[u, pl.ds(16 * k, 16)] = xi * _LPAD + l

    def gcopy0(u):
        return pltpu.make_async_copy(
            q0_hbm.at[qidx_v.at[u]], rows0_v.at[u], sems[u])

    def gcopy1(u):
        return pltpu.make_async_copy(
            q1_hbm.at[qidx_v.at[u]], rows1_v.at[u], sems[u])

    _NBUF = 20
    for u in range(_NBUF):
        prep_idx(u, u)
        gcopy0(u).start()
        gcopy1(u).start()

    def lbody(i, accs):
        for u in range(_NBUF):
            l = _NBUF * i + u
            gcopy0(u).wait()
            gcopy1(u).wait()
            r0 = [rows0_v[u, pl.ds(16 * k, 16)] for k in range(8)]
            r1 = [rows1_v[u, pl.ds(16 * k, 16)] for k in range(8)]

            @pl.when(l + _NBUF < _LEN)
            def _():
                prep_idx(l + _NBUF, u)
                gcopy0(u).start()
                gcopy1(u).start()

            new = []
            for k in range(8):
                new.append(accs[2 * k] + r0[k])
                new.append(accs[2 * k + 1] + r1[k])
            accs = tuple(new)
        return accs

    zero = jnp.zeros((16,), jnp.float32)
    accs = lax.fori_loop(0, _LEN // _NBUF, lbody, (zero,) * 16)

    for k in range(8):
        rows0_v[0, pl.ds(16 * k, 16)] = accs[2 * k]
        rows0_v[1, pl.ds(16 * k, 16)] = accs[2 * k + 1]
    pltpu.sync_copy(rows0_v.at[0], out_hbm.at[0, pl.ds(base, _SPW)])
    pltpu.sync_copy(rows0_v.at[1], out_hbm.at[1, pl.ds(base, _SPW)])


_sc_gather = functools.partial(
    pl.kernel,
    out_type=jax.ShapeDtypeStruct((_NCLS, _BATCH), jnp.float32),
    mesh=plsc.VectorSubcoreMesh(core_axis_name="c", subcore_axis_name="s"),
    scratch_types=[
        pltpu.VMEM((_LEN, _SPW), jnp.int32),
        pltpu.VMEM((20, _SPW), jnp.int32),
        pltpu.VMEM((20, _SPW), jnp.float32),
        pltpu.VMEM((20, _SPW), jnp.float32),
    ] + [pltpu.SemaphoreType.DMA] * 20 + [
    ],
)(_sc_body)


def _logsoftmax_body(z_ref, b_ref, o_ref):
    z = z_ref[...] + b_ref[...]
    m = jnp.max(z, axis=-1, keepdims=True)
    e = jnp.exp(z - m)
    lse = m + jnp.log(jnp.sum(e, axis=-1, keepdims=True))
    o_ref[...] = z - lse


def kernel(x, table, W, b):
    xT = x.T
    Wr = W.reshape(_NCLS, _LEN, _DIM)
    w0t = jnp.pad(Wr[0].T, ((0, 0), (0, _LPAD - _LEN)))
    w1t = jnp.pad(Wr[1].T, ((0, 0), (0, _LPAD - _LEN)))
    q0, q1 = _qmatmul(table, w0t, w1t)
    logits = _sc_gather(xT, q0, q1).T
    return pl.pallas_call(
        _logsoftmax_body,
        out_shape=jax.ShapeDtypeStruct((_BATCH, _NCLS), jnp.float32),
    )(logits, b.reshape(1, _NCLS))
